# baseline (speedup 1.0000x reference)
import jax
import jax.numpy as jnp
from jax.experimental import pallas as pl
from jax.experimental.pallas import tpu as pltpu

N = 4096
F = 32
BR = 256


def _layer_kernel(h01, h02, h10, h12, h20, h21,
                  x0, x1, x2, w0, w1, w2, b0, b1, b2,
                  o0, o1, o2):
    prec = jax.lax.Precision.HIGHEST
    u0 = (jnp.dot(h01[...], x1[...], precision=prec)
          + jnp.dot(h02[...], x2[...], precision=prec))
    o0[...] = jnp.dot(u0, w0[...], precision=prec) + b0[...]
    u1 = (jnp.dot(h10[...], x0[...], precision=prec)
          + jnp.dot(h12[...], x2[...], precision=prec))
    o1[...] = jnp.dot(u1, w1[...], precision=prec) + b1[...]
    u2 = (jnp.dot(h20[...], x0[...], precision=prec)
          + jnp.dot(h21[...], x1[...], precision=prec))
    o2[...] = jnp.dot(u2, w2[...], precision=prec) + b2[...]


def _hgnn_layer(H01, H02, H10, H12, H20, H21, x0, x1, x2,
                w0, w1, w2, b0, b1, b2):
    nb = N // BR
    h_spec = pl.BlockSpec((BR, N), lambda r: (r, 0))
    full = pl.BlockSpec((N, F), lambda r: (0, 0))
    w_spec = pl.BlockSpec((F, F), lambda r: (0, 0))
    b_spec = pl.BlockSpec((1, F), lambda r: (0, 0))
    out_spec = pl.BlockSpec((BR, F), lambda r: (r, 0))
    return pl.pallas_call(
        _layer_kernel,
        grid=(nb,),
        in_specs=[h_spec] * 6 + [full] * 3 + [w_spec] * 3 + [b_spec] * 3,
        out_specs=[out_spec] * 3,
        out_shape=[jax.ShapeDtypeStruct((N, F), jnp.float32)] * 3,
        compiler_params=pltpu.CompilerParams(
            dimension_semantics=("arbitrary",),
        ),
    )(H01, H02, H10, H12, H20, H21, x0, x1, x2,
      w0, w1, w2, b0, b1, b2)


def kernel(x0, x1, x2, H01, H02, H10, H12, H20, H21,
           W1_0, b1_0, W1_1, b1_1, W1_2, b1_2,
           W2_0, b2_0, W2_1, b2_1, W2_2, b2_2):
    b1 = [b.reshape(1, F) for b in (b1_0, b1_1, b1_2)]
    b2 = [b.reshape(1, F) for b in (b2_0, b2_1, b2_2)]
    h0, h1, h2 = _hgnn_layer(H01, H02, H10, H12, H20, H21, x0, x1, x2,
                             W1_0, W1_1, W1_2, *b1)
    o0, o1, o2 = _hgnn_layer(H01, H02, H10, H12, H20, H21, h0, h1, h2,
                             W2_0, W2_1, W2_2, *b2)
    return (o0, o1, o2)


# speedup vs baseline: 4.1492x; 4.1492x over previous
import jax
import jax.numpy as jnp
from jax.experimental import pallas as pl
from jax.experimental.pallas import tpu as pltpu

N = 4096
F = 32
BR = 256


def _layer_kernel(h01, h02, h10, h12, h20, h21,
                  x0, x1, x2, w0, w1, w2, b0, b1, b2,
                  o0, o1, o2):
    prec = jax.lax.Precision.DEFAULT
    u0 = (jnp.dot(h01[...], x1[...], precision=prec)
          + jnp.dot(h02[...], x2[...], precision=prec))
    o0[...] = jnp.dot(u0, w0[...], precision=prec) + b0[...]
    u1 = (jnp.dot(h10[...], x0[...], precision=prec)
          + jnp.dot(h12[...], x2[...], precision=prec))
    o1[...] = jnp.dot(u1, w1[...], precision=prec) + b1[...]
    u2 = (jnp.dot(h20[...], x0[...], precision=prec)
          + jnp.dot(h21[...], x1[...], precision=prec))
    o2[...] = jnp.dot(u2, w2[...], precision=prec) + b2[...]


def _hgnn_layer(H01, H02, H10, H12, H20, H21, x0, x1, x2,
                w0, w1, w2, b0, b1, b2):
    nb = N // BR
    h_spec = pl.BlockSpec((BR, N), lambda r: (r, 0))
    full = pl.BlockSpec((N, F), lambda r: (0, 0))
    w_spec = pl.BlockSpec((F, F), lambda r: (0, 0))
    b_spec = pl.BlockSpec((1, F), lambda r: (0, 0))
    out_spec = pl.BlockSpec((BR, F), lambda r: (r, 0))
    return pl.pallas_call(
        _layer_kernel,
        grid=(nb,),
        in_specs=[h_spec] * 6 + [full] * 3 + [w_spec] * 3 + [b_spec] * 3,
        out_specs=[out_spec] * 3,
        out_shape=[jax.ShapeDtypeStruct((N, F), jnp.float32)] * 3,
        compiler_params=pltpu.CompilerParams(
            dimension_semantics=("arbitrary",),
        ),
    )(H01, H02, H10, H12, H20, H21, x0, x1, x2,
      w0, w1, w2, b0, b1, b2)


def kernel(x0, x1, x2, H01, H02, H10, H12, H20, H21,
           W1_0, b1_0, W1_1, b1_1, W1_2, b1_2,
           W2_0, b2_0, W2_1, b2_1, W2_2, b2_2):
    b1 = [b.reshape(1, F) for b in (b1_0, b1_1, b1_2)]
    b2 = [b.reshape(1, F) for b in (b2_0, b2_1, b2_2)]
    h0, h1, h2 = _hgnn_layer(H01, H02, H10, H12, H20, H21, x0, x1, x2,
                             W1_0, W1_1, W1_2, *b1)
    o0, o1, o2 = _hgnn_layer(H01, H02, H10, H12, H20, H21, h0, h1, h2,
                             W2_0, W2_1, W2_2, *b2)
    return (o0, o1, o2)


# speedup vs baseline: 4.2765x; 1.0307x over previous
import jax
import jax.numpy as jnp
from jax.experimental import pallas as pl
from jax.experimental.pallas import tpu as pltpu

N = 4096
F = 32
BR = 128


def _layer_kernel(h01, h02, h10, h12, h20, h21,
                  x0, x1, x2, w0, w1, w2, b0, b1, b2,
                  o0, o1, o2):
    prec = jax.lax.Precision.DEFAULT
    u0 = (jnp.dot(h01[...], x1[...], precision=prec)
          + jnp.dot(h02[...], x2[...], precision=prec))
    o0[...] = jnp.dot(u0, w0[...], precision=prec) + b0[...]
    u1 = (jnp.dot(h10[...], x0[...], precision=prec)
          + jnp.dot(h12[...], x2[...], precision=prec))
    o1[...] = jnp.dot(u1, w1[...], precision=prec) + b1[...]
    u2 = (jnp.dot(h20[...], x0[...], precision=prec)
          + jnp.dot(h21[...], x1[...], precision=prec))
    o2[...] = jnp.dot(u2, w2[...], precision=prec) + b2[...]


def _hgnn_layer(H01, H02, H10, H12, H20, H21, x0, x1, x2,
                w0, w1, w2, b0, b1, b2):
    nb = N // BR
    h_spec = pl.BlockSpec((BR, N), lambda r: (r, 0))
    full = pl.BlockSpec((N, F), lambda r: (0, 0))
    w_spec = pl.BlockSpec((F, F), lambda r: (0, 0))
    b_spec = pl.BlockSpec((1, F), lambda r: (0, 0))
    out_spec = pl.BlockSpec((BR, F), lambda r: (r, 0))
    return pl.pallas_call(
        _layer_kernel,
        grid=(nb,),
        in_specs=[h_spec] * 6 + [full] * 3 + [w_spec] * 3 + [b_spec] * 3,
        out_specs=[out_spec] * 3,
        out_shape=[jax.ShapeDtypeStruct((N, F), jnp.float32)] * 3,
        compiler_params=pltpu.CompilerParams(
            dimension_semantics=("arbitrary",),
        ),
    )(H01, H02, H10, H12, H20, H21, x0, x1, x2,
      w0, w1, w2, b0, b1, b2)


def kernel(x0, x1, x2, H01, H02, H10, H12, H20, H21,
           W1_0, b1_0, W1_1, b1_1, W1_2, b1_2,
           W2_0, b2_0, W2_1, b2_1, W2_2, b2_2):
    b1 = [b.reshape(1, F) for b in (b1_0, b1_1, b1_2)]
    b2 = [b.reshape(1, F) for b in (b2_0, b2_1, b2_2)]
    h0, h1, h2 = _hgnn_layer(H01, H02, H10, H12, H20, H21, x0, x1, x2,
                             W1_0, W1_1, W1_2, *b1)
    o0, o1, o2 = _hgnn_layer(H01, H02, H10, H12, H20, H21, h0, h1, h2,
                             W2_0, W2_1, W2_2, *b2)
    return (o0, o1, o2)
